# baseline (speedup 1.0000x reference)
import functools

import jax
import jax.numpy as jnp
from jax import lax
from jax.experimental import pallas as pl
from jax.experimental.pallas import tpu as pltpu
from jax.experimental.pallas import tpu_sc as plsc

_E = 64
_K = 8
_TB = 1024
_NC = 2
_NS = 16
_NW = _NC * _NS
_L = 16


def _mm_block(x_ref, w_ref, b_ref, logits_ref, probs_t_ref):
    x = x_ref[...]
    w = w_ref[...]
    l = jnp.dot(x, w, preferred_element_type=jnp.float32) + b_ref[...]
    logits_ref[...] = l
    m = jnp.max(l, axis=1, keepdims=True)
    e = jnp.exp(l - m)
    p = e / jnp.sum(e, axis=1, keepdims=True)
    probs_t_ref[...] = p.T


def _matmul_probs(x2d, W, b2d, chunk, nchunks):
    T, D = x2d.shape
    tch = T // nchunks
    nb = tch // _TB
    off = chunk * nb
    return pl.pallas_call(
        _mm_block,
        grid=(nb,),
        in_specs=[
            pl.BlockSpec((_TB, D), lambda i: (off + i, 0)),
            pl.BlockSpec((D, _E), lambda i: (0, 0)),
            pl.BlockSpec((1, _E), lambda i: (0, 0)),
        ],
        out_specs=[
            pl.BlockSpec((_TB, _E), lambda i: (i, 0)),
            pl.BlockSpec((_E, _TB), lambda i: (0, i)),
        ],
        out_shape=[
            jax.ShapeDtypeStruct((tch, _E), jnp.float32),
            jax.ShapeDtypeStruct((_E, tch), jnp.float32),
        ],
        compiler_params=pltpu.CompilerParams(
            dimension_semantics=("arbitrary",),
        ),
    )(x2d, W, b2d)


def _sc_route_body(tpw, probs_t_hbm, gates_t_hbm, idx_t_hbm, p_v, g_v, i_v):
    wid = lax.axis_index("s") * _NC + lax.axis_index("c")
    base = wid * tpw
    pltpu.sync_copy(probs_t_hbm.at[:, pl.ds(base, tpw)], p_v)

    def group(g, carry):
        off = g * _L
        himask = jnp.full((_L,), 0xFFFFFFC0, jnp.uint32)
        s = [jnp.zeros((_L,), jnp.uint32) for _ in range(_K)]
        for e in range(_E):
            cv = p_v[e, pl.ds(off, _L)]
            cu = lax.bitcast_convert_type(cv, jnp.uint32)
            c = (cu & himask) | jnp.uint32(63 - e)
            for i in range(_K):
                si = s[i]
                s[i] = jnp.maximum(c, si)
                c = jnp.minimum(c, si)
        vals = [lax.bitcast_convert_type(s[i], jnp.float32) for i in range(_K)]
        ids = [
            (jnp.int32(63) - (s[i] & jnp.uint32(63)).astype(jnp.int32))
            for i in range(_K)
        ]
        tot = vals[0]
        for i in range(1, _K):
            tot = tot + vals[i]
        tot = tot + 1e-9
        for i in range(_K):
            g_v[i, pl.ds(off, _L)] = vals[i] / tot
            i_v[i, pl.ds(off, _L)] = ids[i]
        return carry

    lax.fori_loop(0, tpw // _L, group, 0)
    pltpu.sync_copy(g_v, gates_t_hbm.at[:, pl.ds(base, tpw)])
    pltpu.sync_copy(i_v, idx_t_hbm.at[:, pl.ds(base, tpw)])


def _sc_route(probs_t):
    T = probs_t.shape[1]
    tpw = T // _NW
    mesh = plsc.VectorSubcoreMesh(core_axis_name="c", subcore_axis_name="s")
    f = functools.partial(
        pl.kernel,
        mesh=mesh,
        out_type=[
            jax.ShapeDtypeStruct((_K, T), jnp.float32),
            jax.ShapeDtypeStruct((_K, T), jnp.int32),
        ],
        scratch_types=[
            pltpu.VMEM((_E, tpw), jnp.float32),
            pltpu.VMEM((_K, tpw), jnp.float32),
            pltpu.VMEM((_K, tpw), jnp.int32),
        ],
    )(functools.partial(_sc_route_body, tpw))
    return f(probs_t)


_NCHUNKS = 1


@jax.jit
def _router(x2d, W, b2d):
    lg, gs, ix = [], [], []
    for c in range(_NCHUNKS):
        logits_c, probs_t_c = _matmul_probs(x2d, W, b2d, c, _NCHUNKS)
        gates_t_c, idx_t_c = _sc_route(probs_t_c)
        lg.append(logits_c)
        gs.append(gates_t_c.T)
        ix.append(idx_t_c.T)
    if _NCHUNKS == 1:
        return lg[0], gs[0], ix[0]
    logits = jnp.concatenate(lg, axis=0)
    gates = jnp.concatenate(gs, axis=0)
    idx = jnp.concatenate(ix, axis=0)
    return logits, gates, idx


def kernel(x, W, b):
    B, S, D = x.shape
    x2d = x.reshape(B * S, D)
    logits, gates, idx = _router(x2d, W, b.reshape(1, _E))
    return (
        gates.reshape(B, S, _K),
        idx.reshape(B, S, _K),
        logits.reshape(B, S, _E),
    )


# speedup vs baseline: 1.0026x; 1.0026x over previous
import functools

import jax
import jax.numpy as jnp
from jax import lax
from jax.experimental import pallas as pl
from jax.experimental.pallas import tpu as pltpu
from jax.experimental.pallas import tpu_sc as plsc

_E = 64
_K = 8
_TB = 1024
_NC = 2
_NS = 16
_NW = _NC * _NS
_L = 16


def _mm_block(x_ref, w_ref, b_ref, logits_ref, probs_t_ref):
    x = x_ref[...]
    w = w_ref[...]
    l = jnp.dot(x, w, preferred_element_type=jnp.float32) + b_ref[...]
    logits_ref[...] = l
    m = jnp.max(l, axis=1, keepdims=True)
    e = jnp.exp(l - m)
    p = e / jnp.sum(e, axis=1, keepdims=True)
    probs_t_ref[...] = p.T


def _matmul_probs(x2d, W, b2d, chunk, nchunks):
    T, D = x2d.shape
    tch = T // nchunks
    nb = tch // _TB
    off = chunk * nb
    return pl.pallas_call(
        _mm_block,
        grid=(nb,),
        in_specs=[
            pl.BlockSpec((_TB, D), lambda i: (off + i, 0)),
            pl.BlockSpec((D, _E), lambda i: (0, 0)),
            pl.BlockSpec((1, _E), lambda i: (0, 0)),
        ],
        out_specs=[
            pl.BlockSpec((_TB, _E), lambda i: (i, 0)),
            pl.BlockSpec((_E, _TB), lambda i: (0, i)),
        ],
        out_shape=[
            jax.ShapeDtypeStruct((tch, _E), jnp.float32),
            jax.ShapeDtypeStruct((_E, tch), jnp.float32),
        ],
        compiler_params=pltpu.CompilerParams(
            dimension_semantics=("arbitrary",),
        ),
    )(x2d, W, b2d)


def _sc_route_body(tpw, probs_t_hbm, out_t_hbm, o_v, p_v):
    wid = lax.axis_index("s") * _NC + lax.axis_index("c")
    base = wid * tpw
    pltpu.sync_copy(probs_t_hbm.at[:, pl.ds(base, tpw)], p_v)

    def group(g, carry):
        off = g * _L
        himask = jnp.full((_L,), 0xFFFFFFC0, jnp.uint32)
        s = [jnp.zeros((_L,), jnp.uint32) for _ in range(_K)]
        for e in range(_E):
            cv = p_v[e, pl.ds(off, _L)]
            cu = lax.bitcast_convert_type(cv, jnp.uint32)
            c = (cu & himask) | jnp.uint32(63 - e)
            for i in range(_K):
                si = s[i]
                s[i] = jnp.maximum(c, si)
                c = jnp.minimum(c, si)
        vals = [lax.bitcast_convert_type(s[i], jnp.float32) for i in range(_K)]
        ids = [
            (jnp.int32(63) - (s[i] & jnp.uint32(63)).astype(jnp.int32))
            for i in range(_K)
        ]
        tot = vals[0]
        for i in range(1, _K):
            tot = tot + vals[i]
        tot = tot + 1e-9
        for i in range(_K):
            o_v[i, pl.ds(off, _L)] = vals[i] / tot
            o_v[_K + i, pl.ds(off, _L)] = lax.bitcast_convert_type(
                ids[i], jnp.float32)
        return carry

    lax.fori_loop(0, tpw // _L, group, 0)
    pltpu.sync_copy(o_v, out_t_hbm.at[:, pl.ds(base, tpw)])


def _sc_route(probs_t):
    T = probs_t.shape[1]
    tpw = T // _NW
    mesh = plsc.VectorSubcoreMesh(core_axis_name="c", subcore_axis_name="s")
    f = functools.partial(
        pl.kernel,
        mesh=mesh,
        out_type=jax.ShapeDtypeStruct((2 * _K, T), jnp.float32),
        scratch_types=[
            pltpu.VMEM((2 * _K, tpw), jnp.float32),
            pltpu.VMEM((_E, tpw), jnp.float32),
        ],
    )(functools.partial(_sc_route_body, tpw))
    return f(probs_t)


_NCHUNKS = 1


@jax.jit
def _router(x2d, W, b2d):
    logits, probs_t = _matmul_probs(x2d, W, b2d, 0, 1)
    out_t = _sc_route(probs_t)
    out = out_t.T
    gates = out[:, :_K]
    idx = lax.bitcast_convert_type(out[:, _K:], jnp.int32)
    return logits, gates, idx


def kernel(x, W, b):
    B, S, D = x.shape
    x2d = x.reshape(B * S, D)
    logits, gates, idx = _router(x2d, W, b.reshape(1, _E))
    return (
        gates.reshape(B, S, _K),
        idx.reshape(B, S, _K),
        logits.reshape(B, S, _E),
    )


# speedup vs baseline: 1.0249x; 1.0223x over previous
import functools

import jax
import jax.numpy as jnp
from jax import lax
from jax.experimental import pallas as pl
from jax.experimental.pallas import tpu as pltpu
from jax.experimental.pallas import tpu_sc as plsc

_E = 64
_K = 8
_TB = 1024
_NC = 2
_NS = 16
_NW = _NC * _NS
_L = 16


def _mm_block(x_ref, w_ref, b_ref, logits_ref, probs_t_ref):
    x = x_ref[...]
    w = w_ref[...]
    l = jnp.dot(x, w, preferred_element_type=jnp.float32) + b_ref[...]
    logits_ref[...] = l
    m = jnp.max(l, axis=1, keepdims=True)
    e = jnp.exp(l - m)
    p = e / jnp.sum(e, axis=1, keepdims=True)
    probs_t_ref[...] = p.T


def _matmul_probs(x2d, W, b2d, chunk, nchunks):
    T, D = x2d.shape
    tch = T // nchunks
    nb = tch // _TB
    off = chunk * nb
    return pl.pallas_call(
        _mm_block,
        grid=(nb,),
        in_specs=[
            pl.BlockSpec((_TB, D), lambda i: (off + i, 0)),
            pl.BlockSpec((D, _E), lambda i: (0, 0)),
            pl.BlockSpec((1, _E), lambda i: (0, 0)),
        ],
        out_specs=[
            pl.BlockSpec((_TB, _E), lambda i: (i, 0)),
            pl.BlockSpec((_E, _TB), lambda i: (0, i)),
        ],
        out_shape=[
            jax.ShapeDtypeStruct((tch, _E), jnp.float32),
            jax.ShapeDtypeStruct((_E, tch), jnp.float32),
        ],
        compiler_params=pltpu.CompilerParams(
            dimension_semantics=("arbitrary",),
        ),
    )(x2d, W, b2d)


def _sc_route_body(tpw, probs_t_hbm, out_t_hbm, o_v, p_v):
    wid = lax.axis_index("s") * _NC + lax.axis_index("c")
    base = wid * tpw
    pltpu.sync_copy(probs_t_hbm.at[:, pl.ds(base, tpw)], p_v)

    sort8_net = [(0, 1), (2, 3), (4, 5), (6, 7),
                 (0, 2), (1, 3), (4, 6), (5, 7),
                 (1, 2), (5, 6), (0, 4), (3, 7),
                 (1, 5), (2, 6), (1, 4), (3, 6),
                 (2, 4), (3, 5), (3, 4)]
    bitonic8_net = [(0, 4), (1, 5), (2, 6), (3, 7),
                    (0, 2), (1, 3), (4, 6), (5, 7),
                    (0, 1), (2, 3), (4, 5), (6, 7)]

    def ce(v, net):
        for i, j in net:
            hi = jnp.maximum(v[i], v[j])
            v[j] = jnp.minimum(v[i], v[j])
            v[i] = hi
        return v

    def merge_top8(a, b):
        c = [jnp.maximum(a[i], b[_K - 1 - i]) for i in range(_K)]
        return ce(c, bitonic8_net)

    def group(g, carry):
        off = g * _L
        himask = jnp.full((_L,), 0xFFFFFFC0, jnp.uint32)

        def leaf(lf):
            v = []
            for t in range(_K):
                e = lf * _K + t
                cv = p_v[e, pl.ds(off, _L)]
                cu = lax.bitcast_convert_type(cv, jnp.uint32)
                v.append((cu & himask) | jnp.uint32(63 - e))
            return ce(v, sort8_net)

        m01 = merge_top8(leaf(0), leaf(1))
        m23 = merge_top8(leaf(2), leaf(3))
        m03 = merge_top8(m01, m23)
        m45 = merge_top8(leaf(4), leaf(5))
        m67 = merge_top8(leaf(6), leaf(7))
        m47 = merge_top8(m45, m67)
        s = merge_top8(m03, m47)
        vals = [lax.bitcast_convert_type(s[i], jnp.float32) for i in range(_K)]
        ids = [
            (jnp.int32(63) - (s[i] & jnp.uint32(63)).astype(jnp.int32))
            for i in range(_K)
        ]
        tot = vals[0]
        for i in range(1, _K):
            tot = tot + vals[i]
        tot = tot + 1e-9
        for i in range(_K):
            o_v[i, pl.ds(off, _L)] = vals[i] / tot
            o_v[_K + i, pl.ds(off, _L)] = lax.bitcast_convert_type(
                ids[i], jnp.float32)
        return carry

    lax.fori_loop(0, tpw // _L, group, 0)
    pltpu.sync_copy(o_v, out_t_hbm.at[:, pl.ds(base, tpw)])


def _sc_route(probs_t):
    T = probs_t.shape[1]
    tpw = T // _NW
    mesh = plsc.VectorSubcoreMesh(core_axis_name="c", subcore_axis_name="s")
    f = functools.partial(
        pl.kernel,
        mesh=mesh,
        out_type=jax.ShapeDtypeStruct((2 * _K, T), jnp.float32),
        scratch_types=[
            pltpu.VMEM((2 * _K, tpw), jnp.float32),
            pltpu.VMEM((_E, tpw), jnp.float32),
        ],
    )(functools.partial(_sc_route_body, tpw))
    return f(probs_t)


_NCHUNKS = 1


@jax.jit
def _router(x2d, W, b2d):
    logits, probs_t = _matmul_probs(x2d, W, b2d, 0, 1)
    out_t = _sc_route(probs_t)
    out = out_t.T
    gates = out[:, :_K]
    idx = lax.bitcast_convert_type(out[:, _K:], jnp.int32)
    return logits, gates, idx


def kernel(x, W, b):
    B, S, D = x.shape
    x2d = x.reshape(B * S, D)
    logits, gates, idx = _router(x2d, W, b.reshape(1, _E))
    return (
        gates.reshape(B, S, _K),
        idx.reshape(B, S, _K),
        logits.reshape(B, S, _E),
    )
